# Initial kernel scaffold; baseline (speedup 1.0000x reference)
import jax
import jax.numpy as jnp
from jax.experimental import pallas as pl


def kernel(x, labels, center):
    raise NotImplementedError("write your pallas kernel here")


# speedup vs baseline: 4.6673x; 4.6673x over previous
import functools

import jax
import jax.numpy as jnp
from jax import lax
from jax.experimental import pallas as pl
from jax.experimental.pallas import tpu as pltpu
from jax.experimental.pallas import tpu_sc as plsc

_N = 1048576
_FEAT = 64
_CLS = 10000
_CLS_PAD = 10240

_NC = 2
_NS = 16
_SAMPLES_PER_TILE = _N // (_NC * _NS)
_HIST_PER_TILE = _N // _NS
_CHUNK = 128
_NCHUNKS = _SAMPLES_PER_TILE // _CHUNK
_HCHUNK = 4096
_NHCHUNKS = _HIST_PER_TILE // _HCHUNK


def _sc_center_loss(x, labels, center):
    mesh = plsc.VectorSubcoreMesh(core_axis_name="c", subcore_axis_name="s")

    @functools.partial(
        pl.kernel,
        out_type=jax.ShapeDtypeStruct((_NC, 16), jnp.float32),
        mesh=mesh,
        compiler_params=pltpu.CompilerParams(
            needs_layout_passes=False, use_tc_tiling_on_sc=False),
        scratch_types=[
            pltpu.VMEM((_CLS_PAD,), jnp.float32),
            pltpu.VMEM((_HCHUNK,), jnp.int32),
            pltpu.VMEM((16, 640), jnp.float32),
            pltpu.VMEM((_CHUNK,), jnp.int32),
            pltpu.VMEM((_CHUNK, _FEAT), jnp.float32),
            pltpu.VMEM((_CHUNK, _FEAT), jnp.float32),
            pltpu.VMEM((16,), jnp.float32),
            pltpu.VMEM((16, 16), jnp.float32),
            pltpu.VMEM_SHARED((16, _CLS_PAD), jnp.float32),
            pltpu.VMEM_SHARED((16, 16), jnp.float32),
            pltpu.SemaphoreType.DMA,
        ],
    )
    def sc_kernel(x_hbm, labels_hbm, center_hbm, out_hbm,
                  hist_v, lblbuf_v, tmp_v, mlbl_v, x_v, crows_v,
                  accbuf_v, rbuf_v, slab_sp, rslab_sp, gsem):
        c_idx = lax.axis_index("c")
        s_idx = lax.axis_index("s")
        gid = c_idx * _NS + s_idx
        iota16 = lax.iota(jnp.int32, 16)
        ones16 = jnp.ones((16,), jnp.float32)
        zeros16 = jnp.zeros((16,), jnp.float32)

        def zero_body(i, _):
            hist_v[pl.ds(i * 16, 16)] = zeros16
            return 0
        lax.fori_loop(0, _CLS_PAD // 16, zero_body, 0)

        hist_base = s_idx * _HIST_PER_TILE

        def hist_chunk(j, _):
            pltpu.sync_copy(labels_hbm.at[pl.ds(hist_base + j * _HCHUNK,
                                                _HCHUNK)], lblbuf_v)

            def hist_group(g, _):
                lvec = lblbuf_v[pl.ds(g * 16, 16)]
                plsc.addupdate_scatter(hist_v, [lvec], ones16)
                return 0
            lax.fori_loop(0, _HCHUNK // 16, hist_group, 0)
            return 0
        lax.fori_loop(0, _NHCHUNKS, hist_chunk, 0)

        pltpu.sync_copy(hist_v, slab_sp.at[s_idx])
        plsc.subcore_barrier()

        def comb_chunk(cb, _):
            def comb_row(r, _):
                pltpu.sync_copy(slab_sp.at[r, pl.ds(cb * 640, 640)],
                                tmp_v.at[r])
                return 0
            lax.fori_loop(0, 16, comb_row, 0)

            def comb_col(f, _):
                acc = tmp_v[0, pl.ds(f * 16, 16)]
                for r in range(1, 16):
                    acc = acc + tmp_v[r, pl.ds(f * 16, 16)]
                hist_v[pl.ds(cb * 640 + f * 16, 16)] = acc
                return 0
            lax.fori_loop(0, 40, comb_col, 0)
            return 0
        lax.fori_loop(0, 16, comb_chunk, 0)

        main_base = gid * _SAMPLES_PER_TILE

        def chunk_body(ci, acc):
            cb = main_base + ci * _CHUNK
            pltpu.sync_copy(labels_hbm.at[pl.ds(cb, _CHUNK)], mlbl_v)
            gcp = pltpu.async_copy(center_hbm.at[mlbl_v], crows_v, gsem)
            pltpu.sync_copy(x_hbm.at[pl.ds(cb, _CHUNK)], x_v)
            gcp.wait()

            def group_body(g, acc):
                lvec = mlbl_v[pl.ds(g * 16, 16)]
                cnt = plsc.load_gather(hist_v, [lvec])
                rows = iota16 + g * 16
                facc = jnp.zeros((16,), jnp.float32)
                for f in range(_FEAT):
                    colf = jnp.full((16,), f, jnp.int32)
                    xf = plsc.load_gather(x_v, [rows, colf])
                    cf = plsc.load_gather(crows_v, [rows, colf])
                    d = xf - cf
                    facc = facc + d * d
                return acc + facc / cnt
            return lax.fori_loop(0, _CHUNK // 16, group_body, acc)

        acc = lax.fori_loop(0, _NCHUNKS, chunk_body,
                            jnp.zeros((16,), jnp.float32))

        accbuf_v[...] = acc
        pltpu.sync_copy(accbuf_v, rslab_sp.at[s_idx])
        plsc.subcore_barrier()

        @pl.when(s_idx == 0)
        def _():
            pltpu.sync_copy(rslab_sp, rbuf_v)
            tot = rbuf_v[0]
            for r in range(1, 16):
                tot = tot + rbuf_v[r]
            accbuf_v[...] = tot
            pltpu.sync_copy(accbuf_v, out_hbm.at[c_idx])

    return sc_kernel(x, labels, center)


def _tc_sum_kernel(in_ref, o_ref):
    o_ref[0, 0] = jnp.sum(in_ref[...])


def kernel(x, labels, center):
    part = _sc_center_loss(x, labels, center)
    out = pl.pallas_call(
        _tc_sum_kernel,
        out_shape=jax.ShapeDtypeStruct((1, 1), jnp.float32),
        out_specs=pl.BlockSpec(memory_space=pltpu.SMEM),
    )(part)
    return out[0, 0]


# speedup vs baseline: 5.3545x; 1.1472x over previous
import functools

import jax
import jax.numpy as jnp
from jax import lax
from jax.experimental import pallas as pl
from jax.experimental.pallas import tpu as pltpu
from jax.experimental.pallas import tpu_sc as plsc

_N = 1048576
_FEAT = 64
_CLS = 10000
_CLS_PAD = 10240

_NC = 2
_NS = 16
_SAMPLES_PER_TILE = _N // (_NC * _NS)
_CHUNK = 128
_NCHUNKS = _SAMPLES_PER_TILE // _CHUNK
_HROWS = (_N // _NS) // _CHUNK
_HBLK = 32


def _sc_center_loss(x, labels2, center):
    mesh = plsc.VectorSubcoreMesh(core_axis_name="c", subcore_axis_name="s")

    @functools.partial(
        pl.kernel,
        out_type=jax.ShapeDtypeStruct((_NC, 16), jnp.float32),
        mesh=mesh,
        compiler_params=pltpu.CompilerParams(
            needs_layout_passes=False, use_tc_tiling_on_sc=False),
        scratch_types=[
            pltpu.VMEM((_CLS_PAD,), jnp.float32),
            pltpu.VMEM((_HBLK, _CHUNK), jnp.int32),
            pltpu.VMEM((16, 640), jnp.float32),
            pltpu.VMEM((_NCHUNKS, _CHUNK), jnp.int32),
            pltpu.VMEM((2, _CHUNK, _FEAT), jnp.float32),
            pltpu.VMEM((2, _CHUNK, _FEAT), jnp.float32),
            pltpu.VMEM((16,), jnp.float32),
            pltpu.VMEM((16, 16), jnp.float32),
            pltpu.VMEM_SHARED((16, _CLS_PAD), jnp.float32),
            pltpu.VMEM_SHARED((16, 16), jnp.float32),
            pltpu.SemaphoreType.DMA,
            pltpu.SemaphoreType.DMA,
            pltpu.SemaphoreType.DMA,
            pltpu.SemaphoreType.DMA,
        ],
    )
    def sc_kernel(x_hbm, labels2_hbm, center_hbm, out_hbm,
                  hist_v, lblbuf_v, tmp_v, lbl2_v, x2_v, crows2_v,
                  accbuf_v, rbuf_v, slab_sp, rslab_sp,
                  xsem0, xsem1, gsem0, gsem1):
        c_idx = lax.axis_index("c")
        s_idx = lax.axis_index("s")
        gid = c_idx * _NS + s_idx
        iota16 = lax.iota(jnp.int32, 16)
        ones16 = jnp.ones((16,), jnp.float32)
        zeros16 = jnp.zeros((16,), jnp.float32)

        def zero_body(i, _):
            hist_v[pl.ds(i * 16, 16)] = zeros16
            return 0
        lax.fori_loop(0, _CLS_PAD // 16, zero_body, 0)

        hist_row = s_idx * _HROWS

        def hist_chunk(j, _):
            pltpu.sync_copy(labels2_hbm.at[pl.ds(hist_row + j * _HBLK,
                                                 _HBLK)], lblbuf_v)

            def hist_r(r, _):
                def hist_g(g, _):
                    lvec = lblbuf_v[r, pl.ds(g * 16, 16)]
                    plsc.addupdate_scatter(hist_v, [lvec], ones16)
                    return 0
                lax.fori_loop(0, _CHUNK // 16, hist_g, 0)
                return 0
            lax.fori_loop(0, _HBLK, hist_r, 0)
            return 0
        lax.fori_loop(0, _HROWS // _HBLK, hist_chunk, 0)

        pltpu.sync_copy(hist_v, slab_sp.at[s_idx])
        plsc.subcore_barrier()

        def comb_chunk(cb, _):
            def comb_row(r, _):
                pltpu.sync_copy(slab_sp.at[r, pl.ds(cb * 640, 640)],
                                tmp_v.at[r])
                return 0
            lax.fori_loop(0, 16, comb_row, 0)

            def comb_col(f, _):
                acc = tmp_v[0, pl.ds(f * 16, 16)]
                for r in range(1, 16):
                    acc = acc + tmp_v[r, pl.ds(f * 16, 16)]
                hist_v[pl.ds(cb * 640 + f * 16, 16)] = acc
                return 0
            lax.fori_loop(0, 40, comb_col, 0)
            return 0
        lax.fori_loop(0, 16, comb_chunk, 0)

        main_base = gid * _SAMPLES_PER_TILE
        main_row = gid * _NCHUNKS
        pltpu.sync_copy(labels2_hbm.at[pl.ds(main_row, _NCHUNKS)], lbl2_v)

        xsems = (xsem0, xsem1)
        gsems = (gsem0, gsem1)

        def issue(ci, b):
            pltpu.async_copy(center_hbm.at[lbl2_v.at[ci]],
                             crows2_v.at[b], gsems[b])
            pltpu.async_copy(x_hbm.at[pl.ds(main_base + ci * _CHUNK,
                                            _CHUNK)],
                             x2_v.at[b], xsems[b])

        def wait_slot(b):
            pltpu.make_async_copy(x_hbm.at[pl.ds(0, _CHUNK)],
                                  crows2_v.at[b], gsems[b]).wait()
            pltpu.make_async_copy(x_hbm.at[pl.ds(0, _CHUNK)],
                                  x2_v.at[b], xsems[b]).wait()

        def compute(ci, b, acc):
            def group_body(g, acc):
                lvec = lbl2_v[ci, pl.ds(g * 16, 16)]
                cnt = plsc.load_gather(hist_v, [lvec])
                rows = iota16 + g * 16
                facc = jnp.zeros((16,), jnp.float32)
                for f in range(_FEAT):
                    colf = jnp.full((16,), f, jnp.int32)
                    xf = plsc.load_gather(x2_v.at[b], [rows, colf])
                    cf = plsc.load_gather(crows2_v.at[b], [rows, colf])
                    d = xf - cf
                    facc = facc + d * d
                return acc + facc / cnt
            return lax.fori_loop(0, _CHUNK // 16, group_body, acc)

        issue(0, 0)

        def pair_body(p, acc):
            issue(2 * p + 1, 1)
            wait_slot(0)
            acc = compute(2 * p, 0, acc)

            @pl.when(p < _NCHUNKS // 2 - 1)
            def _():
                issue(2 * p + 2, 0)
            wait_slot(1)
            acc = compute(2 * p + 1, 1, acc)
            return acc

        acc = lax.fori_loop(0, _NCHUNKS // 2, pair_body,
                            jnp.zeros((16,), jnp.float32))

        accbuf_v[...] = acc
        pltpu.sync_copy(accbuf_v, rslab_sp.at[s_idx])
        plsc.subcore_barrier()

        @pl.when(s_idx == 0)
        def _():
            pltpu.sync_copy(rslab_sp, rbuf_v)
            tot = rbuf_v[0]
            for r in range(1, 16):
                tot = tot + rbuf_v[r]
            accbuf_v[...] = tot
            pltpu.sync_copy(accbuf_v, out_hbm.at[c_idx])

    return sc_kernel(x, labels2, center)


def _tc_sum_kernel(in_ref, o_ref):
    o_ref[0, 0] = jnp.sum(in_ref[...])


def kernel(x, labels, center):
    labels2 = labels.reshape(_N // _CHUNK, _CHUNK)
    part = _sc_center_loss(x, labels2, center)
    out = pl.pallas_call(
        _tc_sum_kernel,
        out_shape=jax.ShapeDtypeStruct((1, 1), jnp.float32),
        out_specs=pl.BlockSpec(memory_space=pltpu.SMEM),
    )(part)
    return out[0, 0]


# speedup vs baseline: 5.3921x; 1.0070x over previous
import functools

import jax
import jax.numpy as jnp
from jax import lax
from jax.experimental import pallas as pl
from jax.experimental.pallas import tpu as pltpu
from jax.experimental.pallas import tpu_sc as plsc

_N = 1048576
_FEAT = 64
_CLS = 10000
_CLS_PAD = 10240

_NC = 2
_NS = 16
_SAMPLES_PER_TILE = _N // (_NC * _NS)
_CHUNK = 128
_NCHUNKS = _SAMPLES_PER_TILE // _CHUNK
_HROWS = (_N // _NS) // _CHUNK
_HBLK = 32


def _sc_center_loss(x, labels2, center):
    mesh = plsc.VectorSubcoreMesh(core_axis_name="c", subcore_axis_name="s")

    @functools.partial(
        pl.kernel,
        out_type=jax.ShapeDtypeStruct((_NC, 16), jnp.float32),
        mesh=mesh,
        compiler_params=pltpu.CompilerParams(
            needs_layout_passes=False, use_tc_tiling_on_sc=False),
        scratch_types=[
            pltpu.VMEM((_CLS_PAD,), jnp.float32),
            pltpu.VMEM((_HBLK, _CHUNK), jnp.int32),
            pltpu.VMEM((16, 640), jnp.float32),
            pltpu.VMEM((_NCHUNKS // 2, _CHUNK), jnp.int32),
            pltpu.VMEM((2, _CHUNK, _FEAT), jnp.float32),
            pltpu.VMEM((2, _CHUNK, _FEAT), jnp.float32),
            pltpu.VMEM((16,), jnp.float32),
            pltpu.VMEM((16, 16), jnp.float32),
            pltpu.VMEM_SHARED((16, _CLS_PAD), jnp.float32),
            pltpu.VMEM_SHARED((16, 16), jnp.float32),
            pltpu.VMEM_SHARED((_CLS, _FEAT), jnp.float32),
            pltpu.SemaphoreType.DMA,
            pltpu.SemaphoreType.DMA,
            pltpu.SemaphoreType.DMA,
            pltpu.SemaphoreType.DMA,
        ],
    )
    def sc_kernel(x_hbm, labels2_hbm, center_hbm, out_hbm,
                  hist_v, lblbuf_v, tmp_v, lbl2_v, x2_v, crows2_v,
                  accbuf_v, rbuf_v, slab_sp, rslab_sp, center_sp,
                  xsem0, xsem1, gsem0, gsem1):
        c_idx = lax.axis_index("c")
        s_idx = lax.axis_index("s")
        gid = c_idx * _NS + s_idx
        iota16 = lax.iota(jnp.int32, 16)
        ones16 = jnp.ones((16,), jnp.float32)
        zeros16 = jnp.zeros((16,), jnp.float32)

        @pl.when(s_idx == 0)
        def _():
            pltpu.sync_copy(center_hbm, center_sp)

        def zero_body(i, _):
            hist_v[pl.ds(i * 16, 16)] = zeros16
            return 0
        lax.fori_loop(0, _CLS_PAD // 16, zero_body, 0)

        hist_row = s_idx * _HROWS

        def hist_chunk(j, _):
            pltpu.sync_copy(labels2_hbm.at[pl.ds(hist_row + j * _HBLK,
                                                 _HBLK)], lblbuf_v)

            def hist_r(r, _):
                def hist_g(g, _):
                    lvec = lblbuf_v[r, pl.ds(g * 16, 16)]
                    plsc.addupdate_scatter(hist_v, [lvec], ones16)
                    return 0
                lax.fori_loop(0, _CHUNK // 16, hist_g, 0)
                return 0
            lax.fori_loop(0, _HBLK, hist_r, 0)
            return 0
        lax.fori_loop(0, _HROWS // _HBLK, hist_chunk, 0)

        pltpu.sync_copy(hist_v, slab_sp.at[s_idx])
        plsc.subcore_barrier()

        def comb_chunk(cb, _):
            def comb_row(r, _):
                pltpu.sync_copy(slab_sp.at[r, pl.ds(cb * 640, 640)],
                                tmp_v.at[r])
                return 0
            lax.fori_loop(0, 16, comb_row, 0)

            def comb_col(f, _):
                acc = tmp_v[0, pl.ds(f * 16, 16)]
                for r in range(1, 16):
                    acc = acc + tmp_v[r, pl.ds(f * 16, 16)]
                hist_v[pl.ds(cb * 640 + f * 16, 16)] = acc
                return 0
            lax.fori_loop(0, 40, comb_col, 0)
            return 0
        lax.fori_loop(0, 16, comb_chunk, 0)

        main_base = gid * _SAMPLES_PER_TILE
        main_row = gid * _NCHUNKS
        halfchunks = _NCHUNKS // 2

        xsems = (xsem0, xsem1)
        gsems = (gsem0, gsem1)

        def wait_slot(b):
            pltpu.make_async_copy(x_hbm.at[pl.ds(0, _CHUNK)],
                                  crows2_v.at[b], gsems[b]).wait()
            pltpu.make_async_copy(x_hbm.at[pl.ds(0, _CHUNK)],
                                  x2_v.at[b], xsems[b]).wait()

        def half_body(h, acc):
            sbase = main_base + h * halfchunks * _CHUNK
            pltpu.sync_copy(labels2_hbm.at[pl.ds(main_row + h * halfchunks,
                                                 halfchunks)], lbl2_v)

            def issue(cl, b):
                pltpu.async_copy(center_sp.at[lbl2_v.at[cl]],
                                 crows2_v.at[b], gsems[b])
                pltpu.async_copy(x_hbm.at[pl.ds(sbase + cl * _CHUNK,
                                                _CHUNK)],
                                 x2_v.at[b], xsems[b])

            def compute(cl, b, acc):
                def group_body(g, acc):
                    lvec = lbl2_v[cl, pl.ds(g * 16, 16)]
                    cnt = plsc.load_gather(hist_v, [lvec])
                    rows = iota16 + g * 16
                    facc = jnp.zeros((16,), jnp.float32)
                    for f in range(_FEAT):
                        colf = jnp.full((16,), f, jnp.int32)
                        xf = plsc.load_gather(x2_v.at[b], [rows, colf])
                        cf = plsc.load_gather(crows2_v.at[b], [rows, colf])
                        d = xf - cf
                        facc = facc + d * d
                    return acc + facc / cnt
                return lax.fori_loop(0, _CHUNK // 16, group_body, acc)

            issue(0, 0)

            def pair_body(p, acc):
                issue(2 * p + 1, 1)
                wait_slot(0)
                acc = compute(2 * p, 0, acc)

                @pl.when(p < halfchunks // 2 - 1)
                def _():
                    issue(2 * p + 2, 0)
                wait_slot(1)
                acc = compute(2 * p + 1, 1, acc)
                return acc

            return lax.fori_loop(0, halfchunks // 2, pair_body, acc)

        acc = lax.fori_loop(0, 2, half_body, jnp.zeros((16,), jnp.float32))

        accbuf_v[...] = acc
        pltpu.sync_copy(accbuf_v, rslab_sp.at[s_idx])
        plsc.subcore_barrier()

        @pl.when(s_idx == 0)
        def _():
            pltpu.sync_copy(rslab_sp, rbuf_v)
            tot = rbuf_v[0]
            for r in range(1, 16):
                tot = tot + rbuf_v[r]
            accbuf_v[...] = tot
            pltpu.sync_copy(accbuf_v, out_hbm.at[c_idx])

    return sc_kernel(x, labels2, center)


def _tc_sum_kernel(in_ref, o_ref):
    o_ref[0, 0] = jnp.sum(in_ref[...])


def kernel(x, labels, center):
    labels2 = labels.reshape(_N // _CHUNK, _CHUNK)
    part = _sc_center_loss(x, labels2, center)
    out = pl.pallas_call(
        _tc_sum_kernel,
        out_shape=jax.ShapeDtypeStruct((1, 1), jnp.float32),
        out_specs=pl.BlockSpec(memory_space=pltpu.SMEM),
    )(part)
    return out[0, 0]


# speedup vs baseline: 14.3211x; 2.6560x over previous
import functools

import jax
import jax.numpy as jnp
from jax import lax
from jax.experimental import pallas as pl
from jax.experimental.pallas import tpu as pltpu
from jax.experimental.pallas import tpu_sc as plsc

_N = 1048576
_FEAT = 64
_CLS = 10000
_CLS_PAD = 10240

_NC = 2
_NS = 16
_SAMPLES_PER_TILE = _N // (_NC * _NS)
_CHUNK = 128
_NCHUNKS = _SAMPLES_PER_TILE // _CHUNK
_HROWS = (_N // _NS) // _CHUNK
_HBLK = 32


def _sc_center_loss(x, labels2, center):
    mesh = plsc.VectorSubcoreMesh(core_axis_name="c", subcore_axis_name="s")

    @functools.partial(
        pl.kernel,
        out_type=jax.ShapeDtypeStruct((_NC, 16), jnp.float32),
        mesh=mesh,
        compiler_params=pltpu.CompilerParams(
            needs_layout_passes=False, use_tc_tiling_on_sc=False),
        scratch_types=[
            pltpu.VMEM((_CLS_PAD,), jnp.float32),
            pltpu.VMEM((_HBLK, _CHUNK), jnp.int32),
            pltpu.VMEM((16, 640), jnp.float32),
            pltpu.VMEM((_NCHUNKS // 2, _CHUNK), jnp.int32),
            pltpu.VMEM((2, _CHUNK, _FEAT), jnp.float32),
            pltpu.VMEM((2, _CHUNK, _FEAT), jnp.float32),
            pltpu.VMEM((16,), jnp.float32),
            pltpu.VMEM((16, 16), jnp.float32),
            pltpu.VMEM_SHARED((16, _CLS_PAD), jnp.float32),
            pltpu.VMEM_SHARED((16, 16), jnp.float32),
            pltpu.VMEM_SHARED((_CLS, _FEAT), jnp.float32),
            pltpu.SemaphoreType.DMA,
            pltpu.SemaphoreType.DMA,
            pltpu.SemaphoreType.DMA,
            pltpu.SemaphoreType.DMA,
        ],
    )
    def sc_kernel(x_hbm, labels2_hbm, center_hbm, out_hbm,
                  hist_v, lblbuf_v, tmp_v, lbl2_v, x2_v, crows2_v,
                  accbuf_v, rbuf_v, slab_sp, rslab_sp, center_sp,
                  xsem0, xsem1, gsem0, gsem1):
        c_idx = lax.axis_index("c")
        s_idx = lax.axis_index("s")
        gid = c_idx * _NS + s_idx
        iota16 = lax.iota(jnp.int32, 16)
        ones16 = jnp.ones((16,), jnp.float32)
        zeros16 = jnp.zeros((16,), jnp.float32)

        @pl.when(s_idx == 0)
        def _():
            pltpu.sync_copy(center_hbm, center_sp)

        def zero_body(i, _):
            hist_v[pl.ds(i * 16, 16)] = zeros16
            return 0
        lax.fori_loop(0, _CLS_PAD // 16, zero_body, 0)

        hist_row = s_idx * _HROWS

        def hist_chunk(j, _):
            pltpu.sync_copy(labels2_hbm.at[pl.ds(hist_row + j * _HBLK,
                                                 _HBLK)], lblbuf_v)

            def hist_r(r, _):
                def hist_g(g, _):
                    lvec = lblbuf_v[r, pl.ds(g * 16, 16)]
                    plsc.addupdate_scatter(hist_v, [lvec], ones16)
                    return 0
                lax.fori_loop(0, _CHUNK // 16, hist_g, 0)
                return 0
            lax.fori_loop(0, _HBLK, hist_r, 0)
            return 0
        lax.fori_loop(0, _HROWS // _HBLK, hist_chunk, 0)

        pltpu.sync_copy(hist_v, slab_sp.at[s_idx])
        plsc.subcore_barrier()

        def comb_chunk(cb, _):
            def comb_row(r, _):
                pltpu.sync_copy(slab_sp.at[r, pl.ds(cb * 640, 640)],
                                tmp_v.at[r])
                return 0
            lax.fori_loop(0, 16, comb_row, 0)

            def comb_col(f, _):
                acc = tmp_v[0, pl.ds(f * 16, 16)]
                for r in range(1, 16):
                    acc = acc + tmp_v[r, pl.ds(f * 16, 16)]
                hist_v[pl.ds(cb * 640 + f * 16, 16)] = acc
                return 0
            lax.fori_loop(0, 40, comb_col, 0)
            return 0
        lax.fori_loop(0, 16, comb_chunk, 0)

        main_base = gid * _SAMPLES_PER_TILE
        main_row = gid * _NCHUNKS
        halfchunks = _NCHUNKS // 2

        xsems = (xsem0, xsem1)
        gsems = (gsem0, gsem1)

        def wait_slot(b):
            pltpu.make_async_copy(x_hbm.at[pl.ds(0, _CHUNK)],
                                  crows2_v.at[b], gsems[b]).wait()
            pltpu.make_async_copy(x_hbm.at[pl.ds(0, _CHUNK)],
                                  x2_v.at[b], xsems[b]).wait()

        def half_body(h, acc):
            sbase = main_base + h * halfchunks * _CHUNK
            pltpu.sync_copy(labels2_hbm.at[pl.ds(main_row + h * halfchunks,
                                                 halfchunks)], lbl2_v)

            def issue(cl, b):
                pltpu.async_copy(center_sp.at[lbl2_v.at[cl]],
                                 crows2_v.at[b], gsems[b])
                pltpu.async_copy(x_hbm.at[pl.ds(sbase + cl * _CHUNK,
                                                _CHUNK)],
                                 x2_v.at[b], xsems[b])

            def compute(cl, b, acc):
                def group_body(g, acc):
                    lvec = lbl2_v[cl, pl.ds(g * 16, 16)]
                    cnt = plsc.load_gather(hist_v, [lvec])
                    rows = iota16 + g * 16
                    facc = jnp.zeros((16,), jnp.float32)
                    for p in range(_FEAT):
                        colp = (iota16 + p) & (_FEAT - 1)
                        xf = plsc.load_gather(x2_v.at[b], [rows, colp])
                        cf = plsc.load_gather(crows2_v.at[b], [rows, colp])
                        d = xf - cf
                        facc = facc + d * d
                    return acc + facc / cnt
                return lax.fori_loop(0, _CHUNK // 16, group_body, acc)

            issue(0, 0)

            def pair_body(p, acc):
                issue(2 * p + 1, 1)
                wait_slot(0)
                acc = compute(2 * p, 0, acc)

                @pl.when(p < halfchunks // 2 - 1)
                def _():
                    issue(2 * p + 2, 0)
                wait_slot(1)
                acc = compute(2 * p + 1, 1, acc)
                return acc

            return lax.fori_loop(0, halfchunks // 2, pair_body, acc)

        acc = lax.fori_loop(0, 2, half_body, jnp.zeros((16,), jnp.float32))

        accbuf_v[...] = acc
        pltpu.sync_copy(accbuf_v, rslab_sp.at[s_idx])
        plsc.subcore_barrier()

        @pl.when(s_idx == 0)
        def _():
            pltpu.sync_copy(rslab_sp, rbuf_v)
            tot = rbuf_v[0]
            for r in range(1, 16):
                tot = tot + rbuf_v[r]
            accbuf_v[...] = tot
            pltpu.sync_copy(accbuf_v, out_hbm.at[c_idx])

    return sc_kernel(x, labels2, center)


def _tc_sum_kernel(in_ref, o_ref):
    o_ref[0, 0] = jnp.sum(in_ref[...])


def kernel(x, labels, center):
    labels2 = labels.reshape(_N // _CHUNK, _CHUNK)
    part = _sc_center_loss(x, labels2, center)
    out = pl.pallas_call(
        _tc_sum_kernel,
        out_shape=jax.ShapeDtypeStruct((1, 1), jnp.float32),
        out_specs=pl.BlockSpec(memory_space=pltpu.SMEM),
    )(part)
    return out[0, 0]
